# baseline (speedup 1.0000x reference)
import functools

import jax
import jax.numpy as jnp
import numpy as np
from jax import lax
from jax.experimental import pallas as pl
from jax.experimental.pallas import tpu as pltpu
from jax.experimental.pallas import tpu_sc as plsc

N = 10000
E = 320000
D = 128
NE = 10
NB = 8
G = 16
NEN = 3
RMAX = 5.0
AVG = 32.0

NW = 32
NPW = 320
NPAD = NW * NPW
CH = 128
EPA = E + 1024
M2 = EPA // 512
CHA = 2000
NCH = E // CHA
ST = 2560


def _wid():
    return lax.axis_index("s") * 2 + lax.axis_index("c")


def _s_body(dstb, srcb, sh1, sh2, sh3, rf0, rf1, nf0, nf1, meta,
            agg, meta_v, dst_v, src_v, sh1_v, sh2_v, sh3_v, r_v, nf_v,
            acc, sem):
    pltpu.sync_copy(meta, meta_v.at[pl.ds(0, 64)])
    w = _wid()
    cnt = meta_v[pl.ds(w, 16)][0]
    off = meta_v[pl.ds(32 + w, 16)][0]
    n0 = w * NPW
    ntr = (cnt + CH - 1) // CH
    for h in range(2):
        nft = nf0 if h == 0 else nf1
        rft = rf0 if h == 0 else rf1

        def zero(j, _):
            acc[pl.ds(j * 16, 16)] = jnp.zeros((16,), jnp.float32)
            return 0
        lax.fori_loop(0, NPW * 4 * 64 // 16, zero, 0)

        def chunk(t, _):
            base = pl.multiple_of(off + t * CH, 8)
            pltpu.sync_copy(dstb.at[pl.ds(base, CH)], dst_v.at[pl.ds(0, CH)])
            pltpu.sync_copy(srcb.at[pl.ds(base, CH)], src_v)
            pltpu.sync_copy(sh1.at[pl.ds(base, CH)], sh1_v.at[pl.ds(0, CH)])
            pltpu.sync_copy(sh2.at[pl.ds(base, CH)], sh2_v.at[pl.ds(0, CH)])
            pltpu.sync_copy(sh3.at[pl.ds(base, CH)], sh3_v.at[pl.ds(0, CH)])
            pltpu.sync_copy(rft.at[pl.ds(base * 64, CH * 64)], r_v)
            for q in range(CH // 16):
                v = src_v[pl.ds(q * 16, 16)]
                src_v[pl.ds(q * 16, 16)] = jnp.minimum(jnp.maximum(v, 0), N - 1)
            pltpu.async_copy(nft.at[src_v], nf_v, sem).wait()
            nin = jnp.minimum(cnt - t * CH, CH)

            def edge(e, _):
                drow = dst_v[pl.ds(e, 16)][0] - n0
                s1 = sh1_v[pl.ds(e, 16)][0]
                s2 = sh2_v[pl.ds(e, 16)][0]
                s3 = sh3_v[pl.ds(e, 16)][0]
                rbase = e * 64
                abase = drow * 256
                for k4 in range(4):
                    nfv = nf_v[e, pl.ds(k4 * 16, 16)]
                    rv = r_v[pl.ds(rbase + k4 * 16, 16)]
                    m = nfv * rv
                    plsc.addupdate(acc.at[pl.ds(abase + k4 * 16, 16)], m)
                    plsc.addupdate(acc.at[pl.ds(abase + 64 + k4 * 16, 16)], m * s1)
                    plsc.addupdate(acc.at[pl.ds(abase + 128 + k4 * 16, 16)], m * s2)
                    plsc.addupdate(acc.at[pl.ds(abase + 192 + k4 * 16, 16)], m * s3)
                return 0
            lax.fori_loop(0, nin, edge, 0)
            return 0
        lax.fori_loop(0, ntr, chunk, 0)
        pltpu.sync_copy(acc, agg.at[h, pl.ds(n0 * 256, NPW * 256)])


def _s_call(dstb, srcb, sh1, sh2, sh3, rf0, rf1, nf0, nf1, meta):
    return pl.kernel(
        _s_body,
        out_type=jax.ShapeDtypeStruct((2, NPAD * 256), jnp.float32),
        mesh=plsc.VectorSubcoreMesh(core_axis_name="c", subcore_axis_name="s"),
        compiler_params=pltpu.CompilerParams(use_tc_tiling_on_sc=False),
        scratch_types=[
            pltpu.VMEM((80,), jnp.int32),
            pltpu.VMEM((CH + 16,), jnp.int32),
            pltpu.VMEM((CH,), jnp.int32),
            pltpu.VMEM((CH + 16,), jnp.float32),
            pltpu.VMEM((CH + 16,), jnp.float32),
            pltpu.VMEM((CH + 16,), jnp.float32),
            pltpu.VMEM((CH * 64,), jnp.float32),
            pltpu.VMEM((CH, 64), jnp.float32),
            pltpu.VMEM((NPW * 4 * 64,), jnp.float32),
            pltpu.SemaphoreType.DMA,
        ],
    )(dstb, srcb, sh1, sh2, sh3, rf0, rf1, nf0, nf1, meta)


def _a1_body(dst, wids, cnts, din, tmp, wv_ref):
    w = _wid()
    wo = pl.multiple_of(w * 16, 8)
    pltpu.sync_copy(wids.at[pl.ds(wo, 16)], wv_ref)
    wv = wv_ref[pl.ds(0, 16)]
    tmp[pl.ds(0, 16)] = jnp.zeros((16,), jnp.int32)

    def count_chunk(t, _):
        pltpu.sync_copy(dst.at[pl.ds(t * CHA, CHA)], din)

        def inner(q, _):
            v = din[pl.ds(q * 16, 16)]
            b = lax.shift_right_logical(v * 52429, 24)
            m = b == wv
            tmp[pl.ds(0, 16)] = tmp[pl.ds(0, 16)] + jnp.where(m, 1, 0)
            return 0
        lax.fori_loop(0, CHA // 16, inner, 0)
        return 0
    lax.fori_loop(0, NCH, count_chunk, 0)
    pltpu.sync_copy(tmp, cnts.at[pl.ds(wo, 16)])


def a1_call(dst, wids):
    return pl.kernel(
        _a1_body,
        out_type=jax.ShapeDtypeStruct((NW * 16,), jnp.int32),
        mesh=plsc.VectorSubcoreMesh(core_axis_name="c", subcore_axis_name="s"),
        compiler_params=pltpu.CompilerParams(use_tc_tiling_on_sc=False),
        scratch_types=[
            pltpu.VMEM((CHA,), jnp.int32),
            pltpu.VMEM((16,), jnp.int32),
            pltpu.VMEM((16,), jnp.int32),
        ],
    )(dst, wids)


def _a2_body(src, dst, px, py, pz, offs, wids,
             srcb, dstb, dxb, dyb, dzb,
             off_v, din, sin, pos_x, pos_y, pos_z,
             st_src, st_dst, st_dx, st_dy, st_dz, wv_ref):
    w = _wid()
    wo = pl.multiple_of(w * 16, 8)
    pltpu.sync_copy(wids.at[pl.ds(wo, 16)], wv_ref)
    wv = wv_ref[pl.ds(0, 16)]
    pltpu.sync_copy(offs, off_v.at[pl.ds(0, NW * 16)])
    off = off_v[pl.ds(wo, 16)][0]
    off8 = pl.multiple_of(off, 8)
    pltpu.sync_copy(px, pos_x)
    pltpu.sync_copy(py, pos_y)
    pltpu.sync_copy(pz, pos_z)

    def fill_chunk(t, carry):
        p0, g0 = carry
        pltpu.sync_copy(dst.at[pl.ds(t * CHA, CHA)], din)
        pltpu.sync_copy(src.at[pl.ds(t * CHA, CHA)], sin)

        def inner(q, p):
            dv = din[pl.ds(q * 16, 16)]
            sv = sin[pl.ds(q * 16, 16)]
            b = lax.shift_right_logical(dv * 52429, 24)
            m = b == wv
            npop = jnp.sum(jnp.where(m, 1, 0), axis=0)
            pxs = plsc.load_gather(pos_x, [sv])
            pys = plsc.load_gather(pos_y, [sv])
            pzs = plsc.load_gather(pos_z, [sv])
            pxd = plsc.load_gather(pos_x, [dv])
            pyd = plsc.load_gather(pos_y, [dv])
            pzd = plsc.load_gather(pos_z, [dv])
            plsc.store_compressed(st_src.at[pl.ds(p, 16)], sv, mask=m)
            plsc.store_compressed(st_dst.at[pl.ds(p, 16)], dv, mask=m)
            plsc.store_compressed(st_dx.at[pl.ds(p, 16)], pxd - pxs, mask=m)
            plsc.store_compressed(st_dy.at[pl.ds(p, 16)], pyd - pys, mask=m)
            plsc.store_compressed(st_dz.at[pl.ds(p, 16)], pzd - pzs, mask=m)
            return p + npop

        p = lax.fori_loop(0, CHA // 16, inner, p0)

        nblk = p // 256

        def flush_blk(j, _):
            so = pl.multiple_of(j * 256, 8)
            dsto = pl.multiple_of(off8 + g0 + j * 256, 8)
            pltpu.sync_copy(st_src.at[pl.ds(so, 256)], srcb.at[pl.ds(dsto, 256)])
            pltpu.sync_copy(st_dst.at[pl.ds(so, 256)], dstb.at[pl.ds(dsto, 256)])
            pltpu.sync_copy(st_dx.at[pl.ds(so, 256)], dxb.at[pl.ds(dsto, 256)])
            pltpu.sync_copy(st_dy.at[pl.ds(so, 256)], dyb.at[pl.ds(dsto, 256)])
            pltpu.sync_copy(st_dz.at[pl.ds(so, 256)], dzb.at[pl.ds(dsto, 256)])
            return 0
        lax.fori_loop(0, nblk, flush_blk, 0)
        rem = p - nblk * 256

        def move(k, _):
            sk = nblk * 256 + k * 16
            st_src[pl.ds(k * 16, 16)] = st_src[pl.ds(sk, 16)]
            st_dst[pl.ds(k * 16, 16)] = st_dst[pl.ds(sk, 16)]
            st_dx[pl.ds(k * 16, 16)] = st_dx[pl.ds(sk, 16)]
            st_dy[pl.ds(k * 16, 16)] = st_dy[pl.ds(sk, 16)]
            st_dz[pl.ds(k * 16, 16)] = st_dz[pl.ds(sk, 16)]
            return 0
        lax.fori_loop(0, (rem + 15) // 16, move, 0)
        return rem, g0 + nblk * 256

    p, g = lax.fori_loop(0, NCH, fill_chunk, (jnp.int32(0), jnp.int32(0)))

    def tail(j, _):
        j16 = pl.multiple_of(j * 16, 8)
        dsto = pl.multiple_of(off8 + g + j16, 8)
        pltpu.sync_copy(st_src.at[pl.ds(j16, 16)], srcb.at[pl.ds(dsto, 16)])
        pltpu.sync_copy(st_dst.at[pl.ds(j16, 16)], dstb.at[pl.ds(dsto, 16)])
        pltpu.sync_copy(st_dx.at[pl.ds(j16, 16)], dxb.at[pl.ds(dsto, 16)])
        pltpu.sync_copy(st_dy.at[pl.ds(j16, 16)], dyb.at[pl.ds(dsto, 16)])
        pltpu.sync_copy(st_dz.at[pl.ds(j16, 16)], dzb.at[pl.ds(dsto, 16)])
        return 0
    lax.fori_loop(0, (p + 15) // 16, tail, 0)


def a2_call(src, dst, px, py, pz, offs, wids):
    f32, i32 = jnp.float32, jnp.int32
    return pl.kernel(
        _a2_body,
        out_type=[jax.ShapeDtypeStruct((EPA,), i32),
                  jax.ShapeDtypeStruct((EPA,), i32),
                  jax.ShapeDtypeStruct((EPA,), f32),
                  jax.ShapeDtypeStruct((EPA,), f32),
                  jax.ShapeDtypeStruct((EPA,), f32)],
        mesh=plsc.VectorSubcoreMesh(core_axis_name="c", subcore_axis_name="s"),
        compiler_params=pltpu.CompilerParams(use_tc_tiling_on_sc=False),
        scratch_types=[
            pltpu.VMEM((NW * 16 + 16,), i32),
            pltpu.VMEM((CHA,), i32),
            pltpu.VMEM((CHA,), i32),
            pltpu.VMEM((N,), f32),
            pltpu.VMEM((N,), f32),
            pltpu.VMEM((N,), f32),
            pltpu.VMEM((ST,), i32),
            pltpu.VMEM((ST,), i32),
            pltpu.VMEM((ST,), f32),
            pltpu.VMEM((ST,), f32),
            pltpu.VMEM((ST,), f32),
            pltpu.VMEM((16,), i32),
        ],
    )(src, dst, px, py, pz, offs, wids)


def _b_body(dx, dy, dz, w10, w20, w30, w11, w21, w31,
            sh1, sh2, sh3, r0, r1):
    x, y, z = dx[0], dy[0], dz[0]
    s = x * x + y * y + z * z + 1e-12
    ln = jnp.sqrt(s)
    inv = 1.0 / ln
    sq3 = np.float32(np.sqrt(3.0))
    sh1[0] = sq3 * x * inv
    sh2[0] = sq3 * y * inv
    sh3[0] = sq3 * z * inv
    freqs = ((lax.broadcasted_iota(jnp.int32, (NB, 1), 0) + 1
              ).astype(jnp.float32) * np.float32(np.pi / RMAX))
    bes = np.float32(np.sqrt(2.0 / RMAX)) * jnp.sin(freqs * ln) * inv
    xc = ln * np.float32(1.0 / RMAX)
    x5 = xc * xc * xc * xc * xc
    poly = 1.0 - 21.0 * x5 + 35.0 * x5 * xc - 15.0 * x5 * xc * xc
    cut = jnp.where(xc < 1.0, poly, 0.0)
    ef = bes * cut

    def mlp(w1r, w2r, w3r):
        w1, w2, w3 = w1r[...], w2r[...], w3r[...]
        h = jnp.dot(w1.T, ef, preferred_element_type=jnp.float32)
        h = h * jax.nn.sigmoid(h)
        h = jnp.dot(w2.T, h, preferred_element_type=jnp.float32)
        h = h * jax.nn.sigmoid(h)
        rT = jnp.dot(w3.T, h, preferred_element_type=jnp.float32)
        return rT.T

    ra = mlp(w10, w20, w30)
    rb = mlp(w11, w21, w31)
    r0[0] = ra[:, :64]
    r0[1] = ra[:, 64:]
    r1[0] = rb[:, :64]
    r1[1] = rb[:, 64:]


def _b_call(dx, dy, dz, ws):
    full = lambda *shape: pl.BlockSpec(shape, lambda i: (0,) * len(shape))
    row = pl.BlockSpec((1, 1, 512), lambda i: (i, 0, 0))
    half = pl.BlockSpec((2, 512, 64), lambda i: (0, i, 0))
    return pl.pallas_call(
        _b_body,
        grid=(M2,),
        in_specs=[row, row, row,
                  full(NB, 64), full(64, 64), full(64, D),
                  full(NB, 64), full(64, 64), full(64, D)],
        out_specs=[row, row, row, half, half],
        out_shape=[jax.ShapeDtypeStruct((M2, 1, 512), jnp.float32)] * 3
        + [jax.ShapeDtypeStruct((2, EPA, 64), jnp.float32)] * 2,
    )(dx, dy, dz, *ws)


def _c_body(a0, a1, nf, batchr, w0h, w1h, wread, nfo, nfh, ep):
    i = pl.program_id(0)
    upd = (jnp.dot(a0[...], w0h[...], preferred_element_type=jnp.float32)
           + jnp.dot(a1[...], w1h[...], preferred_element_type=jnp.float32))
    nfn = upd * np.float32(1.0 / AVG) + nf[...]
    nfo[...] = nfn
    nfh[0] = nfn[:, :64]
    nfh[1] = nfn[:, 64:]
    en = jnp.dot(nfn, wread[...], preferred_element_type=jnp.float32)
    maskT = (lax.broadcasted_iota(jnp.int32, (G, 512), 0)
             == batchr[0]).astype(jnp.float32)
    p = jnp.dot(maskT, en, preferred_element_type=jnp.float32)

    @pl.when(i == 0)
    def _():
        ep[...] = jnp.zeros_like(ep)
    ep[...] += p


def _c_call(a0, a1, nf, batchr, w0h, w1h, wread8):
    full = lambda *shape: pl.BlockSpec(shape, lambda i: (0,) * len(shape))
    rows256 = pl.BlockSpec((512, 256), lambda i: (i, 0))
    rows128 = pl.BlockSpec((512, 128), lambda i: (i, 0))
    return pl.pallas_call(
        _c_body,
        grid=(NPAD // 512,),
        in_specs=[rows256, rows256, rows128,
                  pl.BlockSpec((1, 1, 512), lambda i: (i, 0, 0)),
                  full(256, 128), full(256, 128), full(128, 8)],
        out_specs=[rows128, pl.BlockSpec((2, 512, 64), lambda i: (0, i, 0)),
                   full(G, 8)],
        out_shape=[jax.ShapeDtypeStruct((NPAD, 128), jnp.float32),
                   jax.ShapeDtypeStruct((2, NPAD, 64), jnp.float32),
                   jax.ShapeDtypeStruct((G, 8), jnp.float32)],
    )(a0, a1, nf, batchr, w0h, w1h, wread8)


def _p_body(na, we, ae, batchr, nfo, nfh, ep):
    i = pl.program_id(0)
    nfn = jnp.dot(na[...], we[...], preferred_element_type=jnp.float32)
    nfo[...] = nfn
    nfh[0] = nfn[:, :64]
    nfh[1] = nfn[:, 64:]
    e0 = jnp.dot(na[...], ae[...], preferred_element_type=jnp.float32)
    maskT = (lax.broadcasted_iota(jnp.int32, (G, 512), 0)
             == batchr[0]).astype(jnp.float32)
    p = jnp.dot(maskT, e0, preferred_element_type=jnp.float32)

    @pl.when(i == 0)
    def _():
        ep[...] = jnp.zeros_like(ep)
    ep[...] += p


def _p_call(nap, wep, aep, batchr):
    full = lambda *shape: pl.BlockSpec(shape, lambda i: (0,) * len(shape))
    return pl.pallas_call(
        _p_body,
        grid=(NPAD // 512,),
        in_specs=[pl.BlockSpec((512, 16), lambda i: (i, 0)),
                  full(16, 128), full(16, 8),
                  pl.BlockSpec((1, 1, 512), lambda i: (i, 0, 0))],
        out_specs=[pl.BlockSpec((512, 128), lambda i: (i, 0)),
                   pl.BlockSpec((2, 512, 64), lambda i: (0, i, 0)),
                   full(G, 8)],
        out_shape=[jax.ShapeDtypeStruct((NPAD, 128), jnp.float32),
                   jax.ShapeDtypeStruct((2, NPAD, 64), jnp.float32),
                   jax.ShapeDtypeStruct((G, 8), jnp.float32)],
    )(nap, wep, aep, batchr)


def kernel(positions, node_attrs, shifts, atomic_energies, W_embed,
           Wr1_0, Wr2_0, Wr3_0, Wupd_0, Wread_0,
           Wr1_1, Wr2_1, Wr3_1, Wupd_1, Wread_1,
           edge_index, batch):
    src = edge_index[0].astype(jnp.int32)
    dst = edge_index[1].astype(jnp.int32)

    wids = jnp.repeat(jnp.arange(NW, dtype=jnp.int32), 16)
    cnts_fat = a1_call(dst, wids)
    counts = cnts_fat.reshape(NW, 16).sum(axis=-1).astype(jnp.int32)
    cpad = ((counts + 15) // 16) * 16
    offsets = jnp.concatenate([jnp.zeros((1,), jnp.int32),
                               jnp.cumsum(cpad)[:-1].astype(jnp.int32)])
    offs_fat = jnp.repeat(offsets, 16)
    meta = jnp.concatenate([counts, offsets])
    BISECT_A1_ONLY = True
    if BISECT_A1_ONLY:
        bucket = dst // NPW
        coff = jnp.concatenate([jnp.zeros((1,), jnp.int32),
                                jnp.cumsum(counts)[:-1].astype(jnp.int32)])
        order = jnp.argsort(bucket)
        bo = bucket[order]
        pos = offsets[bo] + (jnp.arange(E, dtype=jnp.int32) - coff[bo])

        def scat(x):
            return jnp.zeros((EPA,), x.dtype).at[pos].set(x[order])

        srcb = scat(src)
        dstb = scat(dst)
        vec = positions[dst] - positions[src]
        dxb_ = scat(vec[:, 0])
        dyb_ = scat(vec[:, 1])
        dzb_ = scat(vec[:, 2])
    else:
        srcb, dstb, dxb_, dyb_, dzb_ = a2_call(
            src, dst, positions[:, 0].copy(), positions[:, 1].copy(),
            positions[:, 2].copy(), offs_fat, wids)
    dxb = dxb_.reshape(M2, 1, 512)
    dyb = dyb_.reshape(M2, 1, 512)
    dzb = dzb_.reshape(M2, 1, 512)

    sh1, sh2, sh3, r0h, r1h = _b_call(
        dxb, dyb, dzb,
        (Wr1_0, Wr2_0, Wr3_0, Wr1_1, Wr2_1, Wr3_1))
    sh1f = sh1.reshape(-1)
    sh2f = sh2.reshape(-1)
    sh3f = sh3.reshape(-1)

    nap = jnp.zeros((NPAD, 16), jnp.float32).at[:N, :NE].set(node_attrs)
    wep = jnp.zeros((16, D), jnp.float32).at[:NE].set(W_embed)
    aep = jnp.zeros((16, 8), jnp.float32).at[:NE, 0].set(atomic_energies)
    batchr = jnp.full((NPAD,), 255, jnp.int32).at[:N].set(batch).reshape(
        NPAD // 512, 1, 512)
    nf, nfh, e0p = _p_call(nap, wep, aep, batchr)

    energies = jnp.broadcast_to(e0p[:, :1], (G, NEN))
    for (rh, Wupd, Wread) in ((r0h, Wupd_0, Wread_0), (r1h, Wupd_1, Wread_1)):
        ah = _s_call(dstb, srcb, sh1f, sh2f, sh3f,
                     rh[0].reshape(-1), rh[1].reshape(-1),
                     nfh[0], nfh[1], meta)
        a0 = ah[0].reshape(NPAD, 256)
        a1 = ah[1].reshape(NPAD, 256)
        w4 = Wupd.reshape(4, D, D)
        w0h = w4[:, :64, :].reshape(256, D)
        w1h = w4[:, 64:, :].reshape(256, D)
        wread8 = jnp.concatenate([Wread, jnp.zeros((D, 8 - NEN), jnp.float32)],
                                 axis=1)
        nf, nfh, ep = _c_call(a0, a1, nf, batchr, w0h, w1h, wread8)
        energies = energies + ep[:, :NEN]
    return energies


# speedup vs baseline: 1.0423x; 1.0423x over previous
import functools

import jax
import jax.numpy as jnp
import numpy as np
from jax import lax
from jax.experimental import pallas as pl
from jax.experimental.pallas import tpu as pltpu
from jax.experimental.pallas import tpu_sc as plsc

N = 10000
E = 320000
D = 128
NE = 10
NB = 8
G = 16
NEN = 3
RMAX = 5.0
AVG = 32.0

NW = 32
NPW = 320
NPAD = NW * NPW
CH = 256
EPA = E + 1024
M2 = EPA // 512
CHA = 2000
NCH = E // CHA
ST = 2560


def _wid():
    return lax.axis_index("s") * 2 + lax.axis_index("c")


def _s_body(dstb, srcb, sh1, sh2, sh3, rf0, rf1, nf0, nf1, meta,
            agg, meta_v, dst_v, src_v, sh1_v, sh2_v, sh3_v, r_v, nf_v,
            acc, sem):
    pltpu.sync_copy(meta, meta_v.at[pl.ds(0, 64)])
    w = _wid()
    cnt = meta_v[pl.ds(w, 16)][0]
    off = meta_v[pl.ds(32 + w, 16)][0]
    n0 = w * NPW
    ntr = (cnt + CH - 1) // CH
    for h in range(2):
        nft = nf0 if h == 0 else nf1
        rft = rf0 if h == 0 else rf1

        def zero(j, _):
            acc[pl.ds(j * 16, 16)] = jnp.zeros((16,), jnp.float32)
            return 0
        lax.fori_loop(0, NPW * 4 * 64 // 16, zero, 0)

        def chunk(t, _):
            base = pl.multiple_of(off + t * CH, 8)
            pltpu.sync_copy(dstb.at[pl.ds(base, CH)], dst_v.at[pl.ds(0, CH)])
            pltpu.sync_copy(srcb.at[pl.ds(base, CH)], src_v)
            pltpu.sync_copy(sh1.at[pl.ds(base, CH)], sh1_v.at[pl.ds(0, CH)])
            pltpu.sync_copy(sh2.at[pl.ds(base, CH)], sh2_v.at[pl.ds(0, CH)])
            pltpu.sync_copy(sh3.at[pl.ds(base, CH)], sh3_v.at[pl.ds(0, CH)])
            pltpu.sync_copy(rft.at[pl.ds(base * 64, CH * 64)], r_v)
            for q in range(CH // 16):
                v = src_v[pl.ds(q * 16, 16)]
                src_v[pl.ds(q * 16, 16)] = jnp.minimum(jnp.maximum(v, 0), N - 1)
            d0 = pltpu.async_copy(nft.at[src_v.at[pl.ds(0, 128)]],
                                  nf_v.at[pl.ds(0, 128)], sem)
            d1 = pltpu.async_copy(nft.at[src_v.at[pl.ds(128, 128)]],
                                  nf_v.at[pl.ds(128, 128)], sem)
            d0.wait()
            d1.wait()
            nin = jnp.minimum(cnt - t * CH, CH)

            def edge(e, _):
                drow = dst_v[pl.ds(e, 16)][0] - n0
                s1 = sh1_v[pl.ds(e, 16)][0]
                s2 = sh2_v[pl.ds(e, 16)][0]
                s3 = sh3_v[pl.ds(e, 16)][0]
                rbase = e * 64
                abase = drow * 256
                for k4 in range(4):
                    nfv = nf_v[e, pl.ds(k4 * 16, 16)]
                    rv = r_v[pl.ds(rbase + k4 * 16, 16)]
                    m = nfv * rv
                    plsc.addupdate(acc.at[pl.ds(abase + k4 * 16, 16)], m)
                    plsc.addupdate(acc.at[pl.ds(abase + 64 + k4 * 16, 16)], m * s1)
                    plsc.addupdate(acc.at[pl.ds(abase + 128 + k4 * 16, 16)], m * s2)
                    plsc.addupdate(acc.at[pl.ds(abase + 192 + k4 * 16, 16)], m * s3)
                return 0
            lax.fori_loop(0, nin, edge, 0)
            return 0
        lax.fori_loop(0, ntr, chunk, 0)
        pltpu.sync_copy(acc, agg.at[h, pl.ds(n0 * 256, NPW * 256)])


def _s_call(dstb, srcb, sh1, sh2, sh3, rf0, rf1, nf0, nf1, meta):
    return pl.kernel(
        _s_body,
        out_type=jax.ShapeDtypeStruct((2, NPAD * 256), jnp.float32),
        mesh=plsc.VectorSubcoreMesh(core_axis_name="c", subcore_axis_name="s"),
        compiler_params=pltpu.CompilerParams(use_tc_tiling_on_sc=False),
        scratch_types=[
            pltpu.VMEM((80,), jnp.int32),
            pltpu.VMEM((CH + 16,), jnp.int32),
            pltpu.VMEM((CH,), jnp.int32),
            pltpu.VMEM((CH + 16,), jnp.float32),
            pltpu.VMEM((CH + 16,), jnp.float32),
            pltpu.VMEM((CH + 16,), jnp.float32),
            pltpu.VMEM((CH * 64,), jnp.float32),
            pltpu.VMEM((CH, 64), jnp.float32),
            pltpu.VMEM((NPW * 4 * 64,), jnp.float32),
            pltpu.SemaphoreType.DMA,
        ],
    )(dstb, srcb, sh1, sh2, sh3, rf0, rf1, nf0, nf1, meta)


def _a1_body(dst, wids, cnts, din, tmp, wv_ref):
    w = _wid()
    wo = pl.multiple_of(w * 16, 8)
    pltpu.sync_copy(wids.at[pl.ds(wo, 16)], wv_ref)
    wv = wv_ref[pl.ds(0, 16)]
    tmp[pl.ds(0, 16)] = jnp.zeros((16,), jnp.int32)

    def count_chunk(t, _):
        pltpu.sync_copy(dst.at[pl.ds(t * CHA, CHA)], din)

        def inner(q, _):
            v = din[pl.ds(q * 16, 16)]
            b = lax.shift_right_logical(v * 52429, 24)
            m = b == wv
            tmp[pl.ds(0, 16)] = tmp[pl.ds(0, 16)] + jnp.where(m, 1, 0)
            return 0
        lax.fori_loop(0, CHA // 16, inner, 0)
        return 0
    lax.fori_loop(0, NCH, count_chunk, 0)
    pltpu.sync_copy(tmp, cnts.at[pl.ds(wo, 16)])


def a1_call(dst, wids):
    return pl.kernel(
        _a1_body,
        out_type=jax.ShapeDtypeStruct((NW * 16,), jnp.int32),
        mesh=plsc.VectorSubcoreMesh(core_axis_name="c", subcore_axis_name="s"),
        compiler_params=pltpu.CompilerParams(use_tc_tiling_on_sc=False),
        scratch_types=[
            pltpu.VMEM((CHA,), jnp.int32),
            pltpu.VMEM((16,), jnp.int32),
            pltpu.VMEM((16,), jnp.int32),
        ],
    )(dst, wids)


def _a2_body(src, dst, px, py, pz, offs, wids,
             srcb, dstb, dxb, dyb, dzb,
             off_v, din, sin, pos_x, pos_y, pos_z,
             st_src, st_dst, st_dx, st_dy, st_dz, wv_ref):
    w = _wid()
    wo = pl.multiple_of(w * 16, 8)
    pltpu.sync_copy(wids.at[pl.ds(wo, 16)], wv_ref)
    wv = wv_ref[pl.ds(0, 16)]
    pltpu.sync_copy(offs, off_v.at[pl.ds(0, NW * 16)])
    off = off_v[pl.ds(wo, 16)][0]
    off8 = pl.multiple_of(off, 8)
    pltpu.sync_copy(px, pos_x)
    pltpu.sync_copy(py, pos_y)
    pltpu.sync_copy(pz, pos_z)

    def fill_chunk(t, carry):
        p0, g0 = carry
        pltpu.sync_copy(dst.at[pl.ds(t * CHA, CHA)], din)
        pltpu.sync_copy(src.at[pl.ds(t * CHA, CHA)], sin)

        def inner(q, p):
            dv = din[pl.ds(q * 16, 16)]
            sv = sin[pl.ds(q * 16, 16)]
            b = lax.shift_right_logical(dv * 52429, 24)
            m = b == wv
            npop = jnp.sum(jnp.where(m, 1, 0), axis=0)
            pxs = plsc.load_gather(pos_x, [sv])
            pys = plsc.load_gather(pos_y, [sv])
            pzs = plsc.load_gather(pos_z, [sv])
            pxd = plsc.load_gather(pos_x, [dv])
            pyd = plsc.load_gather(pos_y, [dv])
            pzd = plsc.load_gather(pos_z, [dv])
            plsc.store_compressed(st_src.at[pl.ds(p, 16)], sv, mask=m)
            plsc.store_compressed(st_dst.at[pl.ds(p, 16)], dv, mask=m)
            plsc.store_compressed(st_dx.at[pl.ds(p, 16)], pxd - pxs, mask=m)
            plsc.store_compressed(st_dy.at[pl.ds(p, 16)], pyd - pys, mask=m)
            plsc.store_compressed(st_dz.at[pl.ds(p, 16)], pzd - pzs, mask=m)
            return p + npop

        p = lax.fori_loop(0, CHA // 16, inner, p0)

        nblk = p // 256

        def flush_blk(j, _):
            so = pl.multiple_of(j * 256, 8)
            dsto = pl.multiple_of(off8 + g0 + j * 256, 8)
            pltpu.sync_copy(st_src.at[pl.ds(so, 256)], srcb.at[pl.ds(dsto, 256)])
            pltpu.sync_copy(st_dst.at[pl.ds(so, 256)], dstb.at[pl.ds(dsto, 256)])
            pltpu.sync_copy(st_dx.at[pl.ds(so, 256)], dxb.at[pl.ds(dsto, 256)])
            pltpu.sync_copy(st_dy.at[pl.ds(so, 256)], dyb.at[pl.ds(dsto, 256)])
            pltpu.sync_copy(st_dz.at[pl.ds(so, 256)], dzb.at[pl.ds(dsto, 256)])
            return 0
        lax.fori_loop(0, nblk, flush_blk, 0)
        rem = p - nblk * 256

        def move(k, _):
            sk = nblk * 256 + k * 16
            st_src[pl.ds(k * 16, 16)] = st_src[pl.ds(sk, 16)]
            st_dst[pl.ds(k * 16, 16)] = st_dst[pl.ds(sk, 16)]
            st_dx[pl.ds(k * 16, 16)] = st_dx[pl.ds(sk, 16)]
            st_dy[pl.ds(k * 16, 16)] = st_dy[pl.ds(sk, 16)]
            st_dz[pl.ds(k * 16, 16)] = st_dz[pl.ds(sk, 16)]
            return 0
        lax.fori_loop(0, (rem + 15) // 16, move, 0)
        return rem, g0 + nblk * 256

    p, g = lax.fori_loop(0, NCH, fill_chunk, (jnp.int32(0), jnp.int32(0)))

    def tail(j, _):
        j16 = pl.multiple_of(j * 16, 8)
        dsto = pl.multiple_of(off8 + g + j16, 8)
        pltpu.sync_copy(st_src.at[pl.ds(j16, 16)], srcb.at[pl.ds(dsto, 16)])
        pltpu.sync_copy(st_dst.at[pl.ds(j16, 16)], dstb.at[pl.ds(dsto, 16)])
        pltpu.sync_copy(st_dx.at[pl.ds(j16, 16)], dxb.at[pl.ds(dsto, 16)])
        pltpu.sync_copy(st_dy.at[pl.ds(j16, 16)], dyb.at[pl.ds(dsto, 16)])
        pltpu.sync_copy(st_dz.at[pl.ds(j16, 16)], dzb.at[pl.ds(dsto, 16)])
        return 0
    lax.fori_loop(0, (p + 15) // 16, tail, 0)


def a2_call(src, dst, px, py, pz, offs, wids):
    f32, i32 = jnp.float32, jnp.int32
    return pl.kernel(
        _a2_body,
        out_type=[jax.ShapeDtypeStruct((EPA,), i32),
                  jax.ShapeDtypeStruct((EPA,), i32),
                  jax.ShapeDtypeStruct((EPA,), f32),
                  jax.ShapeDtypeStruct((EPA,), f32),
                  jax.ShapeDtypeStruct((EPA,), f32)],
        mesh=plsc.VectorSubcoreMesh(core_axis_name="c", subcore_axis_name="s"),
        compiler_params=pltpu.CompilerParams(use_tc_tiling_on_sc=False),
        scratch_types=[
            pltpu.VMEM((NW * 16 + 16,), i32),
            pltpu.VMEM((CHA,), i32),
            pltpu.VMEM((CHA,), i32),
            pltpu.VMEM((N,), f32),
            pltpu.VMEM((N,), f32),
            pltpu.VMEM((N,), f32),
            pltpu.VMEM((ST,), i32),
            pltpu.VMEM((ST,), i32),
            pltpu.VMEM((ST,), f32),
            pltpu.VMEM((ST,), f32),
            pltpu.VMEM((ST,), f32),
            pltpu.VMEM((16,), i32),
        ],
    )(src, dst, px, py, pz, offs, wids)


def _b_body(dx, dy, dz, w10, w20, w30, w11, w21, w31,
            sh1, sh2, sh3, r0, r1):
    x, y, z = dx[0], dy[0], dz[0]
    s = x * x + y * y + z * z + 1e-12
    ln = jnp.sqrt(s)
    inv = 1.0 / ln
    sq3 = np.float32(np.sqrt(3.0))
    sh1[0] = sq3 * x * inv
    sh2[0] = sq3 * y * inv
    sh3[0] = sq3 * z * inv
    freqs = ((lax.broadcasted_iota(jnp.int32, (NB, 1), 0) + 1
              ).astype(jnp.float32) * np.float32(np.pi / RMAX))
    bes = np.float32(np.sqrt(2.0 / RMAX)) * jnp.sin(freqs * ln) * inv
    xc = ln * np.float32(1.0 / RMAX)
    x5 = xc * xc * xc * xc * xc
    poly = 1.0 - 21.0 * x5 + 35.0 * x5 * xc - 15.0 * x5 * xc * xc
    cut = jnp.where(xc < 1.0, poly, 0.0)
    ef = bes * cut

    def mlp(w1r, w2r, w3r):
        w1, w2, w3 = w1r[...], w2r[...], w3r[...]
        h = jnp.dot(w1.T, ef, preferred_element_type=jnp.float32)
        h = h * jax.nn.sigmoid(h)
        h = jnp.dot(w2.T, h, preferred_element_type=jnp.float32)
        h = h * jax.nn.sigmoid(h)
        rT = jnp.dot(w3.T, h, preferred_element_type=jnp.float32)
        return rT.T

    ra = mlp(w10, w20, w30)
    rb = mlp(w11, w21, w31)
    r0[0] = ra[:, :64]
    r0[1] = ra[:, 64:]
    r1[0] = rb[:, :64]
    r1[1] = rb[:, 64:]


def _b_call(dx, dy, dz, ws):
    full = lambda *shape: pl.BlockSpec(shape, lambda i: (0,) * len(shape))
    row = pl.BlockSpec((1, 1, 512), lambda i: (i, 0, 0))
    half = pl.BlockSpec((2, 512, 64), lambda i: (0, i, 0))
    return pl.pallas_call(
        _b_body,
        grid=(M2,),
        in_specs=[row, row, row,
                  full(NB, 64), full(64, 64), full(64, D),
                  full(NB, 64), full(64, 64), full(64, D)],
        out_specs=[row, row, row, half, half],
        out_shape=[jax.ShapeDtypeStruct((M2, 1, 512), jnp.float32)] * 3
        + [jax.ShapeDtypeStruct((2, EPA, 64), jnp.float32)] * 2,
    )(dx, dy, dz, *ws)


def _c_body(a0, a1, nf, batchr, w0h, w1h, wread, nfo, nfh, ep):
    i = pl.program_id(0)
    upd = (jnp.dot(a0[...], w0h[...], preferred_element_type=jnp.float32)
           + jnp.dot(a1[...], w1h[...], preferred_element_type=jnp.float32))
    nfn = upd * np.float32(1.0 / AVG) + nf[...]
    nfo[...] = nfn
    nfh[0] = nfn[:, :64]
    nfh[1] = nfn[:, 64:]
    en = jnp.dot(nfn, wread[...], preferred_element_type=jnp.float32)
    maskT = (lax.broadcasted_iota(jnp.int32, (G, 512), 0)
             == batchr[0]).astype(jnp.float32)
    p = jnp.dot(maskT, en, preferred_element_type=jnp.float32)

    @pl.when(i == 0)
    def _():
        ep[...] = jnp.zeros_like(ep)
    ep[...] += p


def _c_call(a0, a1, nf, batchr, w0h, w1h, wread8):
    full = lambda *shape: pl.BlockSpec(shape, lambda i: (0,) * len(shape))
    rows256 = pl.BlockSpec((512, 256), lambda i: (i, 0))
    rows128 = pl.BlockSpec((512, 128), lambda i: (i, 0))
    return pl.pallas_call(
        _c_body,
        grid=(NPAD // 512,),
        in_specs=[rows256, rows256, rows128,
                  pl.BlockSpec((1, 1, 512), lambda i: (i, 0, 0)),
                  full(256, 128), full(256, 128), full(128, 8)],
        out_specs=[rows128, pl.BlockSpec((2, 512, 64), lambda i: (0, i, 0)),
                   full(G, 8)],
        out_shape=[jax.ShapeDtypeStruct((NPAD, 128), jnp.float32),
                   jax.ShapeDtypeStruct((2, NPAD, 64), jnp.float32),
                   jax.ShapeDtypeStruct((G, 8), jnp.float32)],
    )(a0, a1, nf, batchr, w0h, w1h, wread8)


def _p_body(na, we, ae, batchr, nfo, nfh, ep):
    i = pl.program_id(0)
    nfn = jnp.dot(na[...], we[...], preferred_element_type=jnp.float32)
    nfo[...] = nfn
    nfh[0] = nfn[:, :64]
    nfh[1] = nfn[:, 64:]
    e0 = jnp.dot(na[...], ae[...], preferred_element_type=jnp.float32)
    maskT = (lax.broadcasted_iota(jnp.int32, (G, 512), 0)
             == batchr[0]).astype(jnp.float32)
    p = jnp.dot(maskT, e0, preferred_element_type=jnp.float32)

    @pl.when(i == 0)
    def _():
        ep[...] = jnp.zeros_like(ep)
    ep[...] += p


def _p_call(nap, wep, aep, batchr):
    full = lambda *shape: pl.BlockSpec(shape, lambda i: (0,) * len(shape))
    return pl.pallas_call(
        _p_body,
        grid=(NPAD // 512,),
        in_specs=[pl.BlockSpec((512, 16), lambda i: (i, 0)),
                  full(16, 128), full(16, 8),
                  pl.BlockSpec((1, 1, 512), lambda i: (i, 0, 0))],
        out_specs=[pl.BlockSpec((512, 128), lambda i: (i, 0)),
                   pl.BlockSpec((2, 512, 64), lambda i: (0, i, 0)),
                   full(G, 8)],
        out_shape=[jax.ShapeDtypeStruct((NPAD, 128), jnp.float32),
                   jax.ShapeDtypeStruct((2, NPAD, 64), jnp.float32),
                   jax.ShapeDtypeStruct((G, 8), jnp.float32)],
    )(nap, wep, aep, batchr)


def kernel(positions, node_attrs, shifts, atomic_energies, W_embed,
           Wr1_0, Wr2_0, Wr3_0, Wupd_0, Wread_0,
           Wr1_1, Wr2_1, Wr3_1, Wupd_1, Wread_1,
           edge_index, batch):
    src = edge_index[0].astype(jnp.int32)
    dst = edge_index[1].astype(jnp.int32)

    wids = jnp.repeat(jnp.arange(NW, dtype=jnp.int32), 16)
    cnts_fat = a1_call(dst, wids)
    counts = cnts_fat.reshape(NW, 16).sum(axis=-1).astype(jnp.int32)
    cpad = ((counts + 15) // 16) * 16
    offsets = jnp.concatenate([jnp.zeros((1,), jnp.int32),
                               jnp.cumsum(cpad)[:-1].astype(jnp.int32)])
    offs_fat = jnp.repeat(offsets, 16)
    meta = jnp.concatenate([counts, offsets])
    BISECT_A1_ONLY = True
    if BISECT_A1_ONLY:
        bucket = dst // NPW
        coff = jnp.concatenate([jnp.zeros((1,), jnp.int32),
                                jnp.cumsum(counts)[:-1].astype(jnp.int32)])
        order = jnp.argsort(bucket)
        bo = bucket[order]
        pos = offsets[bo] + (jnp.arange(E, dtype=jnp.int32) - coff[bo])

        def scat(x):
            return jnp.zeros((EPA,), x.dtype).at[pos].set(x[order])

        srcb = scat(src)
        dstb = scat(dst)
        vec = positions[dst] - positions[src]
        dxb_ = scat(vec[:, 0])
        dyb_ = scat(vec[:, 1])
        dzb_ = scat(vec[:, 2])
    else:
        srcb, dstb, dxb_, dyb_, dzb_ = a2_call(
            src, dst, positions[:, 0].copy(), positions[:, 1].copy(),
            positions[:, 2].copy(), offs_fat, wids)
    dxb = dxb_.reshape(M2, 1, 512)
    dyb = dyb_.reshape(M2, 1, 512)
    dzb = dzb_.reshape(M2, 1, 512)

    sh1, sh2, sh3, r0h, r1h = _b_call(
        dxb, dyb, dzb,
        (Wr1_0, Wr2_0, Wr3_0, Wr1_1, Wr2_1, Wr3_1))
    sh1f = sh1.reshape(-1)
    sh2f = sh2.reshape(-1)
    sh3f = sh3.reshape(-1)

    nap = jnp.zeros((NPAD, 16), jnp.float32).at[:N, :NE].set(node_attrs)
    wep = jnp.zeros((16, D), jnp.float32).at[:NE].set(W_embed)
    aep = jnp.zeros((16, 8), jnp.float32).at[:NE, 0].set(atomic_energies)
    batchr = jnp.full((NPAD,), 255, jnp.int32).at[:N].set(batch).reshape(
        NPAD // 512, 1, 512)
    nf, nfh, e0p = _p_call(nap, wep, aep, batchr)

    energies = jnp.broadcast_to(e0p[:, :1], (G, NEN))
    for (rh, Wupd, Wread) in ((r0h, Wupd_0, Wread_0), (r1h, Wupd_1, Wread_1)):
        ah = _s_call(dstb, srcb, sh1f, sh2f, sh3f,
                     rh[0].reshape(-1), rh[1].reshape(-1),
                     nfh[0], nfh[1], meta)
        a0 = ah[0].reshape(NPAD, 256)
        a1 = ah[1].reshape(NPAD, 256)
        w4 = Wupd.reshape(4, D, D)
        w0h = w4[:, :64, :].reshape(256, D)
        w1h = w4[:, 64:, :].reshape(256, D)
        wread8 = jnp.concatenate([Wread, jnp.zeros((D, 8 - NEN), jnp.float32)],
                                 axis=1)
        nf, nfh, ep = _c_call(a0, a1, nf, batchr, w0h, w1h, wread8)
        energies = energies + ep[:, :NEN]
    return energies
